# Initial kernel scaffold; baseline (speedup 1.0000x reference)
import jax
import jax.numpy as jnp
from jax.experimental import pallas as pl


def kernel(uid_table, mid_table, cat_table, time_table, uid_batch_ph, mid_batch_ph, cat_batch_ph, mid_his_batch_ph, cat_his_batch_ph, item_user_his_batch_ph, item_user_his_time_ph, item_user_his_mid_batch_ph, item_user_his_cat_batch_ph):
    raise NotImplementedError("write your pallas kernel here")


# speedup vs baseline: 8.3016x; 8.3016x over previous
import jax
import jax.numpy as jnp
from jax import lax
from jax.experimental import pallas as pl
from jax.experimental.pallas import tpu as pltpu
from jax.experimental.pallas import tpu_sc as plsc

B = 1024
E = 16
L = 200
LU = 50
LI = 20
NI = LU * LI
NW = 32
BPW = B // NW


def _sum_rows(buf, n, unroll):
  assert n % unroll == 0
  zeros = jnp.zeros((E,), jnp.float32)
  init = (zeros,) * unroll

  def body(i, accs):
    base = i * unroll
    return tuple(a + buf[base + j] for j, a in enumerate(accs))

  accs = lax.fori_loop(0, n // unroll, body, init)
  r = accs[0]
  for a in accs[1:]:
    r = r + a
  return r


def _tri_seq_kernel(uid_table, mid_table, cat_table, time_table,
                    uid_b, mid_b, cat_b, mid_his, cat_his,
                    iuh_uid, iuh_time, iuh_mid, iuh_cat,
                    out,
                    uidb_i, midb_i, catb_i,
                    mid_his_i, cat_his_i, uid_his_i, time_i,
                    iuh_mid_i, iuh_cat_i,
                    emb_uid, emb_mid, emb_cat,
                    r_mid_his, r_cat_his, r_uid_his, r_time,
                    r_iuh_mid, r_iuh_cat,
                    out_buf):
  wid = lax.axis_index("s") * 2 + lax.axis_index("c")
  base = pl.multiple_of(wid * BPW, BPW)

  pltpu.sync_copy(uid_b.at[pl.ds(base, BPW)], uidb_i)
  pltpu.sync_copy(mid_b.at[pl.ds(base, BPW)], midb_i)
  pltpu.sync_copy(cat_b.at[pl.ds(base, BPW)], catb_i)
  pltpu.sync_copy(mid_his.at[pl.ds(base, BPW)], mid_his_i)
  pltpu.sync_copy(cat_his.at[pl.ds(base, BPW)], cat_his_i)
  pltpu.sync_copy(iuh_uid.at[pl.ds(base, BPW)], uid_his_i)
  pltpu.sync_copy(iuh_time.at[pl.ds(base, BPW)], time_i)

  pltpu.sync_copy(uid_table.at[uidb_i], emb_uid)
  pltpu.sync_copy(mid_table.at[midb_i], emb_mid)
  pltpu.sync_copy(cat_table.at[catb_i], emb_cat)

  def row_body(b, carry):
    pltpu.sync_copy(iuh_mid.at[base + b], iuh_mid_i)
    pltpu.sync_copy(iuh_cat.at[base + b], iuh_cat_i)

    pltpu.sync_copy(mid_table.at[mid_his_i.at[b]], r_mid_his)
    pltpu.sync_copy(cat_table.at[cat_his_i.at[b]], r_cat_his)
    pltpu.sync_copy(uid_table.at[uid_his_i.at[b]], r_uid_his)
    pltpu.sync_copy(time_table.at[time_i.at[b]], r_time)
    pltpu.sync_copy(mid_table.at[iuh_mid_i], r_iuh_mid)
    pltpu.sync_copy(cat_table.at[iuh_cat_i], r_iuh_cat)

    out_buf[b, 0:E] = emb_uid[b]
    out_buf[b, E:2 * E] = emb_mid[b]
    out_buf[b, 2 * E:3 * E] = emb_cat[b]
    out_buf[b, 3 * E:4 * E] = _sum_rows(r_mid_his, L, 8)
    out_buf[b, 4 * E:5 * E] = _sum_rows(r_cat_his, L, 8)
    out_buf[b, 5 * E:6 * E] = _sum_rows(r_iuh_mid, NI, 8)
    out_buf[b, 6 * E:7 * E] = _sum_rows(r_iuh_cat, NI, 8)
    out_buf[b, 7 * E:8 * E] = _sum_rows(r_time, LU, 5)
    out_buf[b, 8 * E:9 * E] = _sum_rows(r_uid_his, LU, 5)
    return carry

  lax.fori_loop(0, BPW, row_body, 0)

  pltpu.sync_copy(out_buf, out.at[pl.ds(base, BPW)])


def kernel(uid_table, mid_table, cat_table, time_table, uid_batch_ph,
           mid_batch_ph, cat_batch_ph, mid_his_batch_ph, cat_his_batch_ph,
           item_user_his_batch_ph, item_user_his_time_ph,
           item_user_his_mid_batch_ph, item_user_his_cat_batch_ph):
  iuh_mid = item_user_his_mid_batch_ph.reshape(B, NI)
  iuh_cat = item_user_his_cat_batch_ph.reshape(B, NI)

  mesh = plsc.VectorSubcoreMesh(core_axis_name="c", subcore_axis_name="s")
  f = pl.kernel(
      _tri_seq_kernel,
      out_type=jax.ShapeDtypeStruct((B, 9 * E), jnp.float32),
      mesh=mesh,
      compiler_params=pltpu.CompilerParams(use_tc_tiling_on_sc=False),
      scratch_types=[
          pltpu.VMEM((BPW,), jnp.int32),
          pltpu.VMEM((BPW,), jnp.int32),
          pltpu.VMEM((BPW,), jnp.int32),
          pltpu.VMEM((BPW, L), jnp.int32),
          pltpu.VMEM((BPW, L), jnp.int32),
          pltpu.VMEM((BPW, LU), jnp.int32),
          pltpu.VMEM((BPW, LU), jnp.int32),
          pltpu.VMEM((NI,), jnp.int32),
          pltpu.VMEM((NI,), jnp.int32),
          pltpu.VMEM((BPW, E), jnp.float32),
          pltpu.VMEM((BPW, E), jnp.float32),
          pltpu.VMEM((BPW, E), jnp.float32),
          pltpu.VMEM((L, E), jnp.float32),
          pltpu.VMEM((L, E), jnp.float32),
          pltpu.VMEM((LU, E), jnp.float32),
          pltpu.VMEM((LU, E), jnp.float32),
          pltpu.VMEM((NI, E), jnp.float32),
          pltpu.VMEM((NI, E), jnp.float32),
          pltpu.VMEM((BPW, 9 * E), jnp.float32),
      ],
  )
  return f(uid_table, mid_table, cat_table, time_table,
           uid_batch_ph, mid_batch_ph, cat_batch_ph,
           mid_his_batch_ph, cat_his_batch_ph,
           item_user_his_batch_ph, item_user_his_time_ph,
           iuh_mid, iuh_cat)


# speedup vs baseline: 9.3110x; 1.1216x over previous
import jax
import jax.numpy as jnp
from jax import lax
from jax.experimental import pallas as pl
from jax.experimental.pallas import tpu as pltpu
from jax.experimental.pallas import tpu_sc as plsc

B = 1024
E = 16
L = 200
LU = 50
LI = 20
NI = LU * LI
NW = 32
BPW = B // NW


def _sum_rows(buf, n, unroll):
  assert n % unroll == 0
  zeros = jnp.zeros((E,), jnp.float32)
  init = (zeros,) * unroll

  def body(i, accs):
    base = i * unroll
    return tuple(a + buf[base + j] for j, a in enumerate(accs))

  accs = lax.fori_loop(0, n // unroll, body, init)
  r = accs[0]
  for a in accs[1:]:
    r = r + a
  return r


def _tri_seq_kernel(uid_table, mid_table, cat_table, time_table,
                    uid_b, mid_b, cat_b, mid_his, cat_his,
                    iuh_uid, iuh_time, iuh_mid, iuh_cat,
                    out,
                    uidb_i, midb_i, catb_i,
                    mid_his_i, cat_his_i, uid_his_i, time_i,
                    iuh_mid_i0, iuh_mid_i1, iuh_cat_i0, iuh_cat_i1,
                    emb_uid, emb_mid, emb_cat,
                    r_mid_his0, r_mid_his1, r_cat_his0, r_cat_his1,
                    r_uid_his0, r_uid_his1, r_time0, r_time1,
                    r_iuh_mid0, r_iuh_mid1, r_iuh_cat0, r_iuh_cat1,
                    out_buf,
                    sem_stage, sem_idx0, sem_idx1, sem_g0, sem_g1):
  wid = lax.axis_index("s") * 2 + lax.axis_index("c")
  base = pl.multiple_of(wid * BPW, BPW)

  iuh_mid_i = (iuh_mid_i0, iuh_mid_i1)
  iuh_cat_i = (iuh_cat_i0, iuh_cat_i1)
  r_mid_his = (r_mid_his0, r_mid_his1)
  r_cat_his = (r_cat_his0, r_cat_his1)
  r_uid_his = (r_uid_his0, r_uid_his1)
  r_time = (r_time0, r_time1)
  r_iuh_mid = (r_iuh_mid0, r_iuh_mid1)
  r_iuh_cat = (r_iuh_cat0, r_iuh_cat1)
  sem_idx = (sem_idx0, sem_idx1)
  sem_g = (sem_g0, sem_g1)

  stage = [
      pltpu.async_copy(uid_b.at[pl.ds(base, BPW)], uidb_i, sem_stage),
      pltpu.async_copy(mid_b.at[pl.ds(base, BPW)], midb_i, sem_stage),
      pltpu.async_copy(cat_b.at[pl.ds(base, BPW)], catb_i, sem_stage),
      pltpu.async_copy(mid_his.at[pl.ds(base, BPW)], mid_his_i, sem_stage),
      pltpu.async_copy(cat_his.at[pl.ds(base, BPW)], cat_his_i, sem_stage),
      pltpu.async_copy(iuh_uid.at[pl.ds(base, BPW)], uid_his_i, sem_stage),
      pltpu.async_copy(iuh_time.at[pl.ds(base, BPW)], time_i, sem_stage),
  ]
  for c in stage:
    c.wait()

  e1 = pltpu.async_copy(uid_table.at[uidb_i], emb_uid, sem_stage)
  e2 = pltpu.async_copy(mid_table.at[midb_i], emb_mid, sem_stage)
  e3 = pltpu.async_copy(cat_table.at[catb_i], emb_cat, sem_stage)
  e1.wait(); e2.wait(); e3.wait()

  def idx_descs(b, s):
    return [
        pltpu.make_async_copy(iuh_mid.at[base + b], iuh_mid_i[s],
                              sem_idx[s]),
        pltpu.make_async_copy(iuh_cat.at[base + b], iuh_cat_i[s],
                              sem_idx[s]),
    ]

  def gather_descs(b, s):
    return [
        pltpu.make_async_copy(mid_table.at[mid_his_i.at[b]],
                              r_mid_his[s], sem_g[s]),
        pltpu.make_async_copy(cat_table.at[cat_his_i.at[b]],
                              r_cat_his[s], sem_g[s]),
        pltpu.make_async_copy(uid_table.at[uid_his_i.at[b]],
                              r_uid_his[s], sem_g[s]),
        pltpu.make_async_copy(time_table.at[time_i.at[b]],
                              r_time[s], sem_g[s]),
        pltpu.make_async_copy(mid_table.at[iuh_mid_i[s]],
                              r_iuh_mid[s], sem_g[s]),
        pltpu.make_async_copy(cat_table.at[iuh_cat_i[s]],
                              r_iuh_cat[s], sem_g[s]),
    ]

  def fire(descs):
    for d in descs:
      d.start()

  def wait(descs):
    for d in descs:
      d.wait()

  def accumulate(b, s):
    out_buf[b, 0:E] = emb_uid[b]
    out_buf[b, E:2 * E] = emb_mid[b]
    out_buf[b, 2 * E:3 * E] = emb_cat[b]
    out_buf[b, 3 * E:4 * E] = _sum_rows(r_mid_his[s], L, 8)
    out_buf[b, 4 * E:5 * E] = _sum_rows(r_cat_his[s], L, 8)
    out_buf[b, 5 * E:6 * E] = _sum_rows(r_iuh_mid[s], NI, 8)
    out_buf[b, 6 * E:7 * E] = _sum_rows(r_iuh_cat[s], NI, 8)
    out_buf[b, 7 * E:8 * E] = _sum_rows(r_time[s], LU, 5)
    out_buf[b, 8 * E:9 * E] = _sum_rows(r_uid_his[s], LU, 5)

  fire(idx_descs(0, 0))
  fire(idx_descs(1, 1))
  wait(idx_descs(0, 0))
  fire(gather_descs(0, 0))

  def row_body(j, carry):
    r0 = j * 2
    wait(idx_descs(r0 + 1, 1))
    fire(gather_descs(r0 + 1, 1))
    wait(gather_descs(r0, 0))
    fire(idx_descs(r0 + 2, 0))
    accumulate(r0, 0)
    wait(idx_descs(r0 + 2, 0))
    fire(gather_descs(r0 + 2, 0))
    wait(gather_descs(r0 + 1, 1))
    fire(idx_descs(r0 + 3, 1))
    accumulate(r0 + 1, 1)
    return carry

  lax.fori_loop(0, BPW // 2 - 1, row_body, 0)

  wait(idx_descs(BPW - 1, 1))
  fire(gather_descs(BPW - 1, 1))
  wait(gather_descs(BPW - 2, 0))
  accumulate(BPW - 2, 0)
  wait(gather_descs(BPW - 1, 1))
  accumulate(BPW - 1, 1)

  pltpu.sync_copy(out_buf, out.at[pl.ds(base, BPW)])


def kernel(uid_table, mid_table, cat_table, time_table, uid_batch_ph,
           mid_batch_ph, cat_batch_ph, mid_his_batch_ph, cat_his_batch_ph,
           item_user_his_batch_ph, item_user_his_time_ph,
           item_user_his_mid_batch_ph, item_user_his_cat_batch_ph):
  iuh_mid = item_user_his_mid_batch_ph.reshape(B, NI)
  iuh_cat = item_user_his_cat_batch_ph.reshape(B, NI)

  mesh = plsc.VectorSubcoreMesh(core_axis_name="c", subcore_axis_name="s")
  f = pl.kernel(
      _tri_seq_kernel,
      out_type=jax.ShapeDtypeStruct((B, 9 * E), jnp.float32),
      mesh=mesh,
      compiler_params=pltpu.CompilerParams(use_tc_tiling_on_sc=False),
      scratch_types=[
          pltpu.VMEM((BPW,), jnp.int32),
          pltpu.VMEM((BPW,), jnp.int32),
          pltpu.VMEM((BPW,), jnp.int32),
          pltpu.VMEM((BPW, L), jnp.int32),
          pltpu.VMEM((BPW, L), jnp.int32),
          pltpu.VMEM((BPW, LU), jnp.int32),
          pltpu.VMEM((BPW, LU), jnp.int32),
          pltpu.VMEM((NI,), jnp.int32),
          pltpu.VMEM((NI,), jnp.int32),
          pltpu.VMEM((NI,), jnp.int32),
          pltpu.VMEM((NI,), jnp.int32),
          pltpu.VMEM((BPW, E), jnp.float32),
          pltpu.VMEM((BPW, E), jnp.float32),
          pltpu.VMEM((BPW, E), jnp.float32),
          pltpu.VMEM((L, E), jnp.float32),
          pltpu.VMEM((L, E), jnp.float32),
          pltpu.VMEM((L, E), jnp.float32),
          pltpu.VMEM((L, E), jnp.float32),
          pltpu.VMEM((LU, E), jnp.float32),
          pltpu.VMEM((LU, E), jnp.float32),
          pltpu.VMEM((LU, E), jnp.float32),
          pltpu.VMEM((LU, E), jnp.float32),
          pltpu.VMEM((NI, E), jnp.float32),
          pltpu.VMEM((NI, E), jnp.float32),
          pltpu.VMEM((NI, E), jnp.float32),
          pltpu.VMEM((NI, E), jnp.float32),
          pltpu.VMEM((BPW, 9 * E), jnp.float32),
          pltpu.SemaphoreType.DMA,
          pltpu.SemaphoreType.DMA,
          pltpu.SemaphoreType.DMA,
          pltpu.SemaphoreType.DMA,
          pltpu.SemaphoreType.DMA,
      ],
  )
  return f(uid_table, mid_table, cat_table, time_table,
           uid_batch_ph, mid_batch_ph, cat_batch_ph,
           mid_his_batch_ph, cat_his_batch_ph,
           item_user_his_batch_ph, item_user_his_time_ph,
           iuh_mid, iuh_cat)


# speedup vs baseline: 9.5006x; 1.0204x over previous
import jax
import jax.numpy as jnp
from jax import lax
from jax.experimental import pallas as pl
from jax.experimental.pallas import tpu as pltpu
from jax.experimental.pallas import tpu_sc as plsc

B = 1024
E = 16
L = 200
LU = 50
LI = 20
NI = LU * LI
NW = 32
BPW = B // NW


def _worker_base():
  wid = lax.axis_index("s") * 2 + lax.axis_index("c")
  return pl.multiple_of(wid * BPW, BPW)


def _sum_rows(buf, n, unroll):
  assert n % unroll == 0
  zeros = jnp.zeros((E,), jnp.float32)
  init = (zeros,) * unroll

  def body(i, accs):
    base = i * unroll
    return tuple(a + buf[base + j] for j, a in enumerate(accs))

  accs = lax.fori_loop(0, n // unroll, body, init)
  r = accs[0]
  for a in accs[1:]:
    r = r + a
  return r


def _fire(descs):
  for d in descs:
    d.start()


def _wait(descs):
  for d in descs:
    d.wait()


def _pipeline(idx_descs, gather_descs, accumulate):
  _fire(idx_descs(0, 0))
  _fire(idx_descs(1, 1))
  _wait(idx_descs(0, 0))
  _fire(gather_descs(0, 0))

  def row_body(j, carry):
    r0 = j * 2
    _wait(idx_descs(r0 + 1, 1))
    _fire(gather_descs(r0 + 1, 1))
    _wait(gather_descs(r0, 0))
    _fire(idx_descs(r0 + 2, 0))
    accumulate(r0, 0)
    _wait(idx_descs(r0 + 2, 0))
    _fire(gather_descs(r0 + 2, 0))
    _wait(gather_descs(r0 + 1, 1))
    _fire(idx_descs(r0 + 3, 1))
    accumulate(r0 + 1, 1)
    return carry

  lax.fori_loop(0, BPW // 2 - 1, row_body, 0)

  _wait(idx_descs(BPW - 1, 1))
  _fire(gather_descs(BPW - 1, 1))
  _wait(gather_descs(BPW - 2, 0))
  accumulate(BPW - 2, 0)
  _wait(gather_descs(BPW - 1, 1))
  accumulate(BPW - 1, 1)


def _mid_kernel(mid_table, mid_b, mid_his, iuh_mid,
                out,
                midb_i, mid_his_i, iuh_mid_i0, iuh_mid_i1,
                emb_mid,
                r_mid_his0, r_mid_his1, r_iuh_mid0, r_iuh_mid1,
                out_buf,
                sem_stage, sem_idx0, sem_idx1, sem_g0, sem_g1):
  base = _worker_base()
  iuh_mid_i = (iuh_mid_i0, iuh_mid_i1)
  r_mid_his = (r_mid_his0, r_mid_his1)
  r_iuh_mid = (r_iuh_mid0, r_iuh_mid1)
  sem_idx = (sem_idx0, sem_idx1)
  sem_g = (sem_g0, sem_g1)

  s1 = pltpu.async_copy(mid_b.at[pl.ds(base, BPW)], midb_i, sem_stage)
  s2 = pltpu.async_copy(mid_his.at[pl.ds(base, BPW)], mid_his_i, sem_stage)
  s1.wait(); s2.wait()
  e1 = pltpu.async_copy(mid_table.at[midb_i], emb_mid, sem_stage)
  e1.wait()

  def idx_descs(b, s):
    return [pltpu.make_async_copy(iuh_mid.at[base + b], iuh_mid_i[s],
                                  sem_idx[s])]

  def gather_descs(b, s):
    return [
        pltpu.make_async_copy(mid_table.at[mid_his_i.at[b]],
                              r_mid_his[s], sem_g[s]),
        pltpu.make_async_copy(mid_table.at[iuh_mid_i[s]],
                              r_iuh_mid[s], sem_g[s]),
    ]

  def accumulate(b, s):
    out_buf[b, 0:E] = emb_mid[b]
    out_buf[b, E:2 * E] = _sum_rows(r_mid_his[s], L, 8)
    out_buf[b, 2 * E:3 * E] = _sum_rows(r_iuh_mid[s], NI, 8)

  _pipeline(idx_descs, gather_descs, accumulate)
  pltpu.sync_copy(out_buf, out.at[pl.ds(base, BPW)])


def _rest_kernel(uid_table, cat_table, time_table,
                 uid_b, cat_b, cat_his, iuh_uid, iuh_time, iuh_cat,
                 out,
                 uidb_i, catb_i, cat_his_i, uid_his_i, time_i,
                 iuh_cat_i0, iuh_cat_i1,
                 emb_uid, emb_cat,
                 r_cat_his0, r_cat_his1, r_uid_his0, r_uid_his1,
                 r_time0, r_time1, r_iuh_cat0, r_iuh_cat1,
                 out_buf,
                 sem_stage, sem_idx0, sem_idx1, sem_g0, sem_g1):
  base = _worker_base()
  iuh_cat_i = (iuh_cat_i0, iuh_cat_i1)
  r_cat_his = (r_cat_his0, r_cat_his1)
  r_uid_his = (r_uid_his0, r_uid_his1)
  r_time = (r_time0, r_time1)
  r_iuh_cat = (r_iuh_cat0, r_iuh_cat1)
  sem_idx = (sem_idx0, sem_idx1)
  sem_g = (sem_g0, sem_g1)

  stage = [
      pltpu.async_copy(uid_b.at[pl.ds(base, BPW)], uidb_i, sem_stage),
      pltpu.async_copy(cat_b.at[pl.ds(base, BPW)], catb_i, sem_stage),
      pltpu.async_copy(cat_his.at[pl.ds(base, BPW)], cat_his_i, sem_stage),
      pltpu.async_copy(iuh_uid.at[pl.ds(base, BPW)], uid_his_i, sem_stage),
      pltpu.async_copy(iuh_time.at[pl.ds(base, BPW)], time_i, sem_stage),
  ]
  for c in stage:
    c.wait()
  e1 = pltpu.async_copy(uid_table.at[uidb_i], emb_uid, sem_stage)
  e2 = pltpu.async_copy(cat_table.at[catb_i], emb_cat, sem_stage)
  e1.wait(); e2.wait()

  def idx_descs(b, s):
    return [pltpu.make_async_copy(iuh_cat.at[base + b], iuh_cat_i[s],
                                  sem_idx[s])]

  def gather_descs(b, s):
    return [
        pltpu.make_async_copy(cat_table.at[cat_his_i.at[b]],
                              r_cat_his[s], sem_g[s]),
        pltpu.make_async_copy(uid_table.at[uid_his_i.at[b]],
                              r_uid_his[s], sem_g[s]),
        pltpu.make_async_copy(time_table.at[time_i.at[b]],
                              r_time[s], sem_g[s]),
        pltpu.make_async_copy(cat_table.at[iuh_cat_i[s]],
                              r_iuh_cat[s], sem_g[s]),
    ]

  def accumulate(b, s):
    out_buf[b, 0:E] = emb_uid[b]
    out_buf[b, E:2 * E] = emb_cat[b]
    out_buf[b, 2 * E:3 * E] = _sum_rows(r_cat_his[s], L, 8)
    out_buf[b, 3 * E:4 * E] = _sum_rows(r_iuh_cat[s], NI, 8)
    out_buf[b, 4 * E:5 * E] = _sum_rows(r_time[s], LU, 5)
    out_buf[b, 5 * E:6 * E] = _sum_rows(r_uid_his[s], LU, 5)

  _pipeline(idx_descs, gather_descs, accumulate)
  pltpu.sync_copy(out_buf, out.at[pl.ds(base, BPW)])


def kernel(uid_table, mid_table, cat_table, time_table, uid_batch_ph,
           mid_batch_ph, cat_batch_ph, mid_his_batch_ph, cat_his_batch_ph,
           item_user_his_batch_ph, item_user_his_time_ph,
           item_user_his_mid_batch_ph, item_user_his_cat_batch_ph):
  iuh_mid = item_user_his_mid_batch_ph.reshape(B, NI)
  iuh_cat = item_user_his_cat_batch_ph.reshape(B, NI)

  mesh = plsc.VectorSubcoreMesh(core_axis_name="c", subcore_axis_name="s")
  params = pltpu.CompilerParams(use_tc_tiling_on_sc=False)

  rest = pl.kernel(
      _rest_kernel,
      out_type=jax.ShapeDtypeStruct((B, 6 * E), jnp.float32),
      mesh=mesh,
      compiler_params=params,
      scratch_types=[
          pltpu.VMEM((BPW,), jnp.int32),
          pltpu.VMEM((BPW,), jnp.int32),
          pltpu.VMEM((BPW, L), jnp.int32),
          pltpu.VMEM((BPW, LU), jnp.int32),
          pltpu.VMEM((BPW, LU), jnp.int32),
          pltpu.VMEM((NI,), jnp.int32),
          pltpu.VMEM((NI,), jnp.int32),
          pltpu.VMEM((BPW, E), jnp.float32),
          pltpu.VMEM((BPW, E), jnp.float32),
          pltpu.VMEM((L, E), jnp.float32),
          pltpu.VMEM((L, E), jnp.float32),
          pltpu.VMEM((LU, E), jnp.float32),
          pltpu.VMEM((LU, E), jnp.float32),
          pltpu.VMEM((LU, E), jnp.float32),
          pltpu.VMEM((LU, E), jnp.float32),
          pltpu.VMEM((NI, E), jnp.float32),
          pltpu.VMEM((NI, E), jnp.float32),
          pltpu.VMEM((BPW, 6 * E), jnp.float32),
          pltpu.SemaphoreType.DMA,
          pltpu.SemaphoreType.DMA,
          pltpu.SemaphoreType.DMA,
          pltpu.SemaphoreType.DMA,
          pltpu.SemaphoreType.DMA,
      ],
  )(uid_table, cat_table, time_table, uid_batch_ph, cat_batch_ph,
    cat_his_batch_ph, item_user_his_batch_ph, item_user_his_time_ph,
    iuh_cat)

  mid = pl.kernel(
      _mid_kernel,
      out_type=jax.ShapeDtypeStruct((B, 3 * E), jnp.float32),
      mesh=mesh,
      compiler_params=params,
      scratch_types=[
          pltpu.VMEM((BPW,), jnp.int32),
          pltpu.VMEM((BPW, L), jnp.int32),
          pltpu.VMEM((NI,), jnp.int32),
          pltpu.VMEM((NI,), jnp.int32),
          pltpu.VMEM((BPW, E), jnp.float32),
          pltpu.VMEM((L, E), jnp.float32),
          pltpu.VMEM((L, E), jnp.float32),
          pltpu.VMEM((NI, E), jnp.float32),
          pltpu.VMEM((NI, E), jnp.float32),
          pltpu.VMEM((BPW, 3 * E), jnp.float32),
          pltpu.SemaphoreType.DMA,
          pltpu.SemaphoreType.DMA,
          pltpu.SemaphoreType.DMA,
          pltpu.SemaphoreType.DMA,
          pltpu.SemaphoreType.DMA,
      ],
  )(mid_table, mid_batch_ph, mid_his_batch_ph, iuh_mid)

  return jnp.concatenate([
      rest[:, 0:E],
      mid[:, 0:E],
      rest[:, E:2 * E],
      mid[:, E:2 * E],
      rest[:, 2 * E:3 * E],
      mid[:, 2 * E:3 * E],
      rest[:, 3 * E:4 * E],
      rest[:, 4 * E:5 * E],
      rest[:, 5 * E:6 * E],
  ], axis=1)


# speedup vs baseline: 12.1202x; 1.2757x over previous
import jax
import jax.numpy as jnp
from jax import lax
from jax.experimental import pallas as pl
from jax.experimental.pallas import tpu as pltpu
from jax.experimental.pallas import tpu_sc as plsc

B = 1024
E = 16
L = 200
LU = 50
LI = 20
NI = LU * LI
NW = 32
BPW = B // NW


def _worker_base():
  wid = lax.axis_index("s") * 2 + lax.axis_index("c")
  return pl.multiple_of(wid * BPW, BPW)


def _sum_rows(buf, start, n, unroll):
  assert n % unroll == 0
  zeros = jnp.zeros((E,), jnp.float32)
  init = (zeros,) * unroll

  def body(i, accs):
    o = start + i * unroll
    return tuple(a + buf[o + j] for j, a in enumerate(accs))

  accs = lax.fori_loop(0, n // unroll, body, init)
  r = accs[0]
  for a in accs[1:]:
    r = r + a
  return r


def _fire(descs):
  for d in descs:
    d.start()


def _wait(descs):
  for d in descs:
    d.wait()


def _pipeline(idx_descs, gather_descs, accumulate):
  _fire(idx_descs(0, 0))
  _fire(idx_descs(1, 1))
  _wait(idx_descs(0, 0))
  _fire(gather_descs(0, 0))

  def row_body(j, carry):
    r0 = j * 2
    _wait(idx_descs(r0 + 1, 1))
    _fire(gather_descs(r0 + 1, 1))
    _wait(gather_descs(r0, 0))
    _fire(idx_descs(r0 + 2, 0))
    accumulate(r0, 0)
    _wait(idx_descs(r0 + 2, 0))
    _fire(gather_descs(r0 + 2, 0))
    _wait(gather_descs(r0 + 1, 1))
    _fire(idx_descs(r0 + 3, 1))
    accumulate(r0 + 1, 1)
    return carry

  lax.fori_loop(0, BPW // 2 - 1, row_body, 0)

  _wait(idx_descs(BPW - 1, 1))
  _fire(gather_descs(BPW - 1, 1))
  _wait(gather_descs(BPW - 2, 0))
  accumulate(BPW - 2, 0)
  _wait(gather_descs(BPW - 1, 1))
  accumulate(BPW - 1, 1)


def _mid_kernel(mid_table, mid_b, mid_his, iuh_mid,
                out,
                midb_i, mid_his_i, iuh_mid_i0, iuh_mid_i1,
                emb_mid,
                r_mid_his0, r_mid_his1, r_iuh_mid0, r_iuh_mid1,
                out_buf,
                sem_stage, sem_idx0, sem_idx1, sem_g0, sem_g1):
  base = _worker_base()
  iuh_mid_i = (iuh_mid_i0, iuh_mid_i1)
  r_mid_his = (r_mid_his0, r_mid_his1)
  r_iuh_mid = (r_iuh_mid0, r_iuh_mid1)
  sem_idx = (sem_idx0, sem_idx1)
  sem_g = (sem_g0, sem_g1)

  s1 = pltpu.async_copy(mid_b.at[pl.ds(base, BPW)], midb_i, sem_stage)
  s2 = pltpu.async_copy(mid_his.at[pl.ds(base, BPW)], mid_his_i, sem_stage)
  s1.wait(); s2.wait()
  e1 = pltpu.async_copy(mid_table.at[midb_i], emb_mid, sem_stage)
  e1.wait()

  def idx_descs(b, s):
    return [pltpu.make_async_copy(iuh_mid.at[base + b], iuh_mid_i[s],
                                  sem_idx[s])]

  def gather_descs(b, s):
    return [
        pltpu.make_async_copy(mid_table.at[mid_his_i.at[b]],
                              r_mid_his[s], sem_g[s]),
        pltpu.make_async_copy(mid_table.at[iuh_mid_i[s]],
                              r_iuh_mid[s], sem_g[s]),
    ]

  def accumulate(b, s):
    out_buf[b, 0:E] = emb_mid[b]
    out_buf[b, E:2 * E] = _sum_rows(r_mid_his[s], 0, L, 8)
    out_buf[b, 2 * E:3 * E] = _sum_rows(r_iuh_mid[s], 0, NI, 8)

  _pipeline(idx_descs, gather_descs, accumulate)
  pltpu.sync_copy(out_buf, out.at[pl.ds(base, BPW)])


def _uid_kernel(uid_table, uid_b, iuh_uid_f,
                out,
                uidb_i, uid_his_i, emb_uid, r_uid_his, out_buf,
                sem_stage, sem_g):
  base = _worker_base()
  s1 = pltpu.async_copy(uid_b.at[pl.ds(base, BPW)], uidb_i, sem_stage)
  s2 = pltpu.async_copy(iuh_uid_f.at[pl.ds(base * LU, BPW * LU)],
                        uid_his_i, sem_stage)
  s1.wait(); s2.wait()
  g1 = pltpu.async_copy(uid_table.at[uidb_i], emb_uid, sem_g)
  g2 = pltpu.async_copy(uid_table.at[uid_his_i], r_uid_his, sem_g)
  g1.wait(); g2.wait()

  def row(b, carry):
    out_buf[b, 0:E] = emb_uid[b]
    out_buf[b, E:2 * E] = _sum_rows(r_uid_his, b * LU, LU, 5)
    return carry

  lax.fori_loop(0, BPW, row, 0)
  pltpu.sync_copy(out_buf, out.at[pl.ds(base, BPW)])


OC2 = 4 * E


def _seg(tab_flat, idx_flat, stride, g, n, out_flat, col0):
  lanes = lax.iota(jnp.int32, E) + g * E
  row_off = lanes * stride
  out_off = lanes * OC2 + col0
  init = (jnp.zeros((E,), jnp.float32),) * E

  def body(l, accs):
    ids = plsc.load_gather(idx_flat, [row_off + l])
    addr = ids * E
    return tuple(a + plsc.load_gather(tab_flat, [addr + e])
                 for e, a in enumerate(accs))

  accs = lax.fori_loop(0, n, body, init)
  for e, a in enumerate(accs):
    plsc.store_scatter(out_flat, [out_off + e], a)


def _cattime_kernel(cat_flat, time_flat,
                    cat_b, cat_his_f, iuh_time_f, iuh_cat_f,
                    out,
                    cat_tab, time_tab, catb_i, cat_his_i, time_i,
                    iuh_cat_i, out_flat,
                    sem_stage):
  base = _worker_base()
  stage = [
      pltpu.async_copy(cat_flat, cat_tab, sem_stage),
      pltpu.async_copy(time_flat, time_tab, sem_stage),
      pltpu.async_copy(cat_b.at[pl.ds(base, BPW)], catb_i, sem_stage),
      pltpu.async_copy(cat_his_f.at[pl.ds(base * L, BPW * L)],
                       cat_his_i, sem_stage),
      pltpu.async_copy(iuh_time_f.at[pl.ds(base * LU, BPW * LU)],
                       time_i, sem_stage),
      pltpu.async_copy(iuh_cat_f.at[pl.ds(base * NI, BPW * NI)],
                       iuh_cat_i, sem_stage),
  ]
  for c in stage:
    c.wait()

  for g in range(BPW // E):
    lanes = lax.iota(jnp.int32, E) + g * E
    ids = plsc.load_gather(catb_i, [lanes])
    addr = ids * E
    out_off = lanes * OC2
    for e in range(E):
      col = plsc.load_gather(cat_tab, [addr + e])
      plsc.store_scatter(out_flat, [out_off + e], col)
    _seg(cat_tab, cat_his_i, L, g, L, out_flat, E)
    _seg(cat_tab, iuh_cat_i, NI, g, NI, out_flat, 2 * E)
    _seg(time_tab, time_i, LU, g, LU, out_flat, 3 * E)

  pltpu.sync_copy(out_flat, out.at[pl.ds(base * OC2, BPW * OC2)])


def kernel(uid_table, mid_table, cat_table, time_table, uid_batch_ph,
           mid_batch_ph, cat_batch_ph, mid_his_batch_ph, cat_his_batch_ph,
           item_user_his_batch_ph, item_user_his_time_ph,
           item_user_his_mid_batch_ph, item_user_his_cat_batch_ph):
  iuh_mid = item_user_his_mid_batch_ph.reshape(B, NI)
  iuh_cat_f = item_user_his_cat_batch_ph.reshape(B * NI)
  iuh_uid_f = item_user_his_batch_ph.reshape(B * LU)
  iuh_time_f = item_user_his_time_ph.reshape(B * LU)
  cat_his_f = cat_his_batch_ph.reshape(B * L)
  cat_flat = cat_table.reshape(1000 * E)
  time_flat = time_table.reshape(16 * E)

  mesh = plsc.VectorSubcoreMesh(core_axis_name="c", subcore_axis_name="s")
  params = pltpu.CompilerParams(use_tc_tiling_on_sc=False)
  ct_params = pltpu.CompilerParams(use_tc_tiling_on_sc=False,
                                   needs_layout_passes=False)

  uid = pl.kernel(
      _uid_kernel,
      out_type=jax.ShapeDtypeStruct((B, 2 * E), jnp.float32),
      mesh=mesh,
      compiler_params=params,
      scratch_types=[
          pltpu.VMEM((BPW,), jnp.int32),
          pltpu.VMEM((BPW * LU,), jnp.int32),
          pltpu.VMEM((BPW, E), jnp.float32),
          pltpu.VMEM((BPW * LU, E), jnp.float32),
          pltpu.VMEM((BPW, 2 * E), jnp.float32),
          pltpu.SemaphoreType.DMA,
          pltpu.SemaphoreType.DMA,
      ],
  )(uid_table, uid_batch_ph, iuh_uid_f)

  cattime = pl.kernel(
      _cattime_kernel,
      out_type=jax.ShapeDtypeStruct((B * OC2,), jnp.float32),
      mesh=mesh,
      compiler_params=ct_params,
      scratch_types=[
          pltpu.VMEM((1000 * E,), jnp.float32),
          pltpu.VMEM((16 * E,), jnp.float32),
          pltpu.VMEM((BPW,), jnp.int32),
          pltpu.VMEM((BPW * L,), jnp.int32),
          pltpu.VMEM((BPW * LU,), jnp.int32),
          pltpu.VMEM((BPW * NI,), jnp.int32),
          pltpu.VMEM((BPW * OC2,), jnp.float32),
          pltpu.SemaphoreType.DMA,
      ],
  )(cat_flat, time_flat, cat_batch_ph, cat_his_f, iuh_time_f, iuh_cat_f)
  cattime = cattime.reshape(B, OC2)

  mid = pl.kernel(
      _mid_kernel,
      out_type=jax.ShapeDtypeStruct((B, 3 * E), jnp.float32),
      mesh=mesh,
      compiler_params=params,
      scratch_types=[
          pltpu.VMEM((BPW,), jnp.int32),
          pltpu.VMEM((BPW, L), jnp.int32),
          pltpu.VMEM((NI,), jnp.int32),
          pltpu.VMEM((NI,), jnp.int32),
          pltpu.VMEM((BPW, E), jnp.float32),
          pltpu.VMEM((L, E), jnp.float32),
          pltpu.VMEM((L, E), jnp.float32),
          pltpu.VMEM((NI, E), jnp.float32),
          pltpu.VMEM((NI, E), jnp.float32),
          pltpu.VMEM((BPW, 3 * E), jnp.float32),
          pltpu.SemaphoreType.DMA,
          pltpu.SemaphoreType.DMA,
          pltpu.SemaphoreType.DMA,
          pltpu.SemaphoreType.DMA,
          pltpu.SemaphoreType.DMA,
      ],
  )(mid_table, mid_batch_ph, mid_his_batch_ph, iuh_mid)

  return jnp.concatenate([
      uid[:, 0:E],
      mid[:, 0:E],
      cattime[:, 0:E],
      mid[:, E:2 * E],
      cattime[:, E:2 * E],
      mid[:, 2 * E:3 * E],
      cattime[:, 2 * E:3 * E],
      cattime[:, 3 * E:4 * E],
      uid[:, E:2 * E],
  ], axis=1)
